# Initial kernel scaffold; baseline (speedup 1.0000x reference)
import jax
import jax.numpy as jnp
from jax.experimental import pallas as pl


def kernel(x, indices, values, bias):
    raise NotImplementedError("write your pallas kernel here")


# speedup vs baseline: 15.9188x; 15.9188x over previous
import jax
import jax.numpy as jnp
from jax import lax
from jax.experimental import pallas as pl
from jax.experimental.pallas import tpu as pltpu
from jax.experimental.pallas import tpu_sc as plsc

N_IN = 16384
N_OUT = 16384
NNZ = 2621440
B = 64
BH = B // 2

NC = 2
NS = 16
NW = NC * NS
K = 1024
QK = 128
NQ = K // QK
PER_W = NNZ // NW
CHUNKS = PER_W // K
ROWS_PER_TILE = N_OUT // NS
ZR = 256


def _sc_body(xt0_hbm, xt1_hbm, col_hbm, row_hbm, val_hbm, out_hbm,
             col_v, row_v, val_v, rows_v, zbuf, acc, sem):
    c = lax.axis_index("c")
    s = lax.axis_index("s")
    wid = s * NC + c

    @pl.loop(0, ZR)
    def _zero(r):
        for blk in range(BH // 16):
            zbuf[r, pl.ds(blk * 16, 16)] = jnp.zeros((16,), jnp.float32)

    for h, xt_hbm in ((0, xt0_hbm), (1, xt1_hbm)):
        for z in range(ROWS_PER_TILE // ZR):
            pltpu.sync_copy(zbuf, acc.at[pl.ds(s * ROWS_PER_TILE + z * ZR, ZR)])
        plsc.subcore_barrier()

        @pl.loop(0, CHUNKS)
        def _chunk(k):
            base = pl.multiple_of(wid * PER_W + k * K, K)
            r0 = pl.multiple_of(base // QK, NQ)
            pltpu.sync_copy(col_hbm.at[pl.ds(r0, NQ)], col_v)
            pltpu.sync_copy(row_hbm.at[pl.ds(r0, NQ)], row_v)
            pltpu.sync_copy(val_hbm.at[pl.ds(base, K)], val_v)

            cps = [
                pltpu.async_copy(xt_hbm.at[col_v.at[q]],
                                 rows_v.at[pl.ds(q * QK, QK)], sem)
                for q in range(NQ)
            ]
            for cp in cps:
                cp.wait()

            @pl.loop(0, K // 16)
            def _scale(g):
                v16 = val_v[pl.ds(g * 16, 16)]
                for jj in range(16):
                    j = g * 16 + jj
                    vj = jnp.broadcast_to(v16[jj], (16,))
                    for blk in range(BH // 16):
                        sl = pl.ds(blk * 16, 16)
                        rows_v[j, sl] = rows_v[j, sl] * vj

            for q in range(NQ):
                pltpu.sync_copy(rows_v.at[pl.ds(q * QK, QK)],
                                acc.at[row_v.at[q]], add=True)

        plsc.subcore_barrier()
        pltpu.sync_copy(
            acc.at[pl.ds(s * ROWS_PER_TILE, ROWS_PER_TILE)],
            out_hbm.at[c].at[h].at[pl.ds(s * ROWS_PER_TILE, ROWS_PER_TILE)])
        plsc.subcore_barrier()


def _spmm_partials(xt0, xt1, col2, row2, val):
    mesh = plsc.VectorSubcoreMesh(core_axis_name="c", subcore_axis_name="s")
    f = pl.kernel(
        _sc_body,
        out_type=jax.ShapeDtypeStruct((NC, 2, N_OUT, BH), jnp.float32),
        mesh=mesh,
        scratch_types=[
            pltpu.VMEM((NQ, QK), jnp.int32),
            pltpu.VMEM((NQ, QK), jnp.int32),
            pltpu.VMEM((K,), jnp.float32),
            pltpu.VMEM((K, BH), jnp.float32),
            pltpu.VMEM((ZR, BH), jnp.float32),
            pltpu.VMEM_SHARED((N_OUT, BH), jnp.float32),
            pltpu.SemaphoreType.DMA,
        ],
        compiler_params=pltpu.CompilerParams(use_tc_tiling_on_sc=False),
    )
    return f(xt0, xt1, col2, row2, val)


RB = 512


def _merge_body(p_ref, b_ref, eye_ref, o_ref):
    for h in range(2):
        ps = p_ref[0, h] + p_ref[1, h] + b_ref[...]
        o_ref[pl.ds(h * BH, BH), :] = lax.dot_general(
            eye_ref[...], ps, (((1,), (1,)), ((), ())),
            preferred_element_type=jnp.float32)


def _merge(partials, bias, eye):
    return pl.pallas_call(
        _merge_body,
        grid=(N_OUT // RB,),
        in_specs=[
            pl.BlockSpec((NC, 2, RB, BH), lambda i: (0, 0, i, 0)),
            pl.BlockSpec((RB, 1), lambda i: (i, 0)),
            pl.BlockSpec((BH, BH), lambda i: (0, 0)),
        ],
        out_specs=pl.BlockSpec((B, RB), lambda i: (0, i)),
        out_shape=jax.ShapeDtypeStruct((B, N_OUT), jnp.float32),
    )(partials, bias, eye)


@jax.jit
def kernel(x, indices, values, bias):
    xt = jnp.swapaxes(x, 0, 1)
    xt0 = xt[:, :BH]
    xt1 = xt[:, BH:]
    row2 = indices[0].reshape(NNZ // QK, QK)
    col2 = indices[1].reshape(NNZ // QK, QK)
    partials = _spmm_partials(xt0, xt1, col2, row2, values)
    eye = jnp.eye(BH, dtype=jnp.float32)
    return _merge(partials, bias, eye)


# speedup vs baseline: 31.5040x; 1.9790x over previous
import jax
import jax.numpy as jnp
from jax import lax
from jax.experimental import pallas as pl
from jax.experimental.pallas import tpu as pltpu
from jax.experimental.pallas import tpu_sc as plsc

N_IN = 16384
N_OUT = 16384
NNZ = 2621440
B = 64
BH = B // 2

NC = 2
NS = 16
NW = NC * NS
K = 1024
QK = 128
NQ = K // QK
PER_W = NNZ // NW
CHUNKS = PER_W // K
NI = 4
SUPER = CHUNKS // NI
ROWS_PER_TILE = N_OUT // NS
ZR = 256


def _sc_body(xt0_hbm, xt1_hbm, col_hbm, row_hbm, val_hbm, out_hbm,
             colb, rowb, valb, rbuf, zbuf, acc, isem, gsem, ssem):
    c = lax.axis_index("c")
    s = lax.axis_index("s")
    wid = s * NC + c

    def idx_start(k, bi):
        base = pl.multiple_of(wid * PER_W + k * K, K)
        r0 = pl.multiple_of(base // QK, NQ)
        pltpu.async_copy(col_hbm.at[pl.ds(r0, NQ)], colb.at[bi], isem)
        pltpu.async_copy(row_hbm.at[pl.ds(r0, NQ)], rowb.at[bi], isem)
        pltpu.async_copy(val_hbm.at[pl.ds(base, K)], valb.at[bi], isem)

    def idx_wait():
        pltpu.make_async_copy(col_hbm.at[pl.ds(0, NQ)], colb.at[0], isem).wait()
        pltpu.make_async_copy(row_hbm.at[pl.ds(0, NQ)], rowb.at[0], isem).wait()
        pltpu.make_async_copy(val_hbm.at[pl.ds(0, K)], valb.at[0], isem).wait()

    def gather_start(xt_hbm, bi, br):
        for q in range(NQ):
            pltpu.async_copy(xt_hbm.at[colb.at[bi].at[q]],
                             rbuf.at[br].at[pl.ds(q * QK, QK)], gsem)

    def gather_wait(xt_hbm):
        for q in range(NQ):
            pltpu.make_async_copy(xt_hbm.at[colb.at[0].at[q]],
                                  rbuf.at[0].at[pl.ds(q * QK, QK)], gsem).wait()

    def scale_scatter(bi, br):
        for q in range(NQ):
            @pl.loop(0, QK // 16)
            def _scale(g):
                off = q * QK + g * 16
                v16 = valb.at[bi][pl.ds(off, 16)]
                for jj in range(16):
                    vj = jnp.broadcast_to(v16[jj], (16,))
                    for blk in range(BH // 16):
                        sl = pl.ds(blk * 16, 16)
                        rbuf.at[br][off + jj, sl] = rbuf.at[br][off + jj, sl] * vj
            pltpu.async_copy(rbuf.at[br].at[pl.ds(q * QK, QK)],
                             acc.at[rowb.at[bi].at[q]], ssem, add=True)

    def scatter_wait():
        for q in range(NQ):
            pltpu.make_async_copy(rbuf.at[0].at[pl.ds(q * QK, QK)],
                                  acc.at[rowb.at[0].at[q]], ssem).wait()

    @pl.loop(0, ZR)
    def _zero(r):
        for blk in range(BH // 16):
            zbuf[r, pl.ds(blk * 16, 16)] = jnp.zeros((16,), jnp.float32)

    for h, xt_hbm in ((0, xt0_hbm), (1, xt1_hbm)):
        for z in range(ROWS_PER_TILE // ZR):
            pltpu.sync_copy(zbuf, acc.at[pl.ds(s * ROWS_PER_TILE + z * ZR, ZR)])
        plsc.subcore_barrier()

        idx_start(0, 0)
        idx_wait()
        gather_start(xt_hbm, 0, 0)
        idx_start(1, 1)

        @pl.loop(0, SUPER)
        def _super(k4):
            not_last = k4 < SUPER - 1
            for b in range(NI):
                k = k4 * NI + b

                if b < NI - 1:
                    idx_wait()
                else:
                    @pl.when(not_last)
                    def _():
                        idx_wait()

                if b == 0:
                    @pl.when(k4 >= 1)
                    def _():
                        scatter_wait()
                else:
                    scatter_wait()

                if b < NI - 1:
                    gather_start(xt_hbm, b + 1, (b + 1) % 2)
                else:
                    @pl.when(not_last)
                    def _():
                        gather_start(xt_hbm, 0, (b + 1) % 2)

                gather_wait(xt_hbm)
                scale_scatter(b, b % 2)

                if b < NI - 2:
                    idx_start(k + 2, b + 2)
                else:
                    @pl.when(not_last)
                    def _():
                        idx_start(k + 2, (b + 2) % NI)

        scatter_wait()

        plsc.subcore_barrier()
        pltpu.sync_copy(
            acc.at[pl.ds(s * ROWS_PER_TILE, ROWS_PER_TILE)],
            out_hbm.at[c].at[h].at[pl.ds(s * ROWS_PER_TILE, ROWS_PER_TILE)])
        plsc.subcore_barrier()


def _spmm_partials(xt0, xt1, col2, row2, val):
    mesh = plsc.VectorSubcoreMesh(core_axis_name="c", subcore_axis_name="s")
    f = pl.kernel(
        _sc_body,
        out_type=jax.ShapeDtypeStruct((NC, 2, N_OUT, BH), jnp.float32),
        mesh=mesh,
        scratch_types=[
            pltpu.VMEM((NI, NQ, QK), jnp.int32),
            pltpu.VMEM((NI, NQ, QK), jnp.int32),
            pltpu.VMEM((NI, K), jnp.float32),
            pltpu.VMEM((2, K, BH), jnp.float32),
            pltpu.VMEM((ZR, BH), jnp.float32),
            pltpu.VMEM_SHARED((N_OUT, BH), jnp.float32),
            pltpu.SemaphoreType.DMA,
            pltpu.SemaphoreType.DMA,
            pltpu.SemaphoreType.DMA,
        ],
        compiler_params=pltpu.CompilerParams(use_tc_tiling_on_sc=False),
    )
    return f(xt0, xt1, col2, row2, val)


RB = 512


def _merge_body(p_ref, b_ref, eye_ref, o_ref):
    for h in range(2):
        ps = p_ref[0, h] + p_ref[1, h] + b_ref[...]
        o_ref[pl.ds(h * BH, BH), :] = lax.dot_general(
            eye_ref[...], ps, (((1,), (1,)), ((), ())),
            preferred_element_type=jnp.float32)


def _merge(partials, bias, eye):
    return pl.pallas_call(
        _merge_body,
        grid=(N_OUT // RB,),
        in_specs=[
            pl.BlockSpec((NC, 2, RB, BH), lambda i: (0, 0, i, 0)),
            pl.BlockSpec((RB, 1), lambda i: (i, 0)),
            pl.BlockSpec((BH, BH), lambda i: (0, 0)),
        ],
        out_specs=pl.BlockSpec((B, RB), lambda i: (0, i)),
        out_shape=jax.ShapeDtypeStruct((B, N_OUT), jnp.float32),
    )(partials, bias, eye)


@jax.jit
def kernel(x, indices, values, bias):
    xt = jnp.swapaxes(x, 0, 1)
    xt0 = xt[:, :BH]
    xt1 = xt[:, BH:]
    row2 = indices[0].reshape(NNZ // QK, QK)
    col2 = indices[1].reshape(NNZ // QK, QK)
    partials = _spmm_partials(xt0, xt1, col2, row2, values)
    eye = jnp.eye(BH, dtype=jnp.float32)
    return _merge(partials, bias, eye)
